# baseline (speedup 1.0000x reference)
import functools

import jax
import jax.numpy as jnp
from jax.experimental import pallas as pl
from jax.experimental.pallas import tpu as pltpu

NEG_SLOPE = 0.01
BN_EPS = 1e-5


def _leaky(v):
    return jnp.where(v >= 0, v, NEG_SLOPE * v)


def _rot(a, s, p):
    s = s % p
    if s == 0:
        return a
    return jnp.concatenate([a[:, s:], a[:, :s]], axis=1)


def _kw_stack(a, p):
    return jnp.concatenate([_rot(a, -1, p), a, _rot(a, 1, p)], axis=0)


def _kh_combine(o, c, p, wp):
    return _rot(o[:c], -wp, p) + o[c:2 * c] + _rot(o[2 * c:], wp, p)


def _stats_kernel(x_ref, wd_ref, xb_ref, s_ref, ss_ref, *, H, W, Wp, P):
    x = x_ref[...]
    d = jnp.dot(wd_ref[...], x, preferred_element_type=jnp.float32)
    s = jnp.sum(d, axis=1, keepdims=True)
    ss = jnp.sum(d * d, axis=1, keepdims=True)
    s_ref[...] = jnp.broadcast_to(s, s_ref.shape)
    ss_ref[...] = jnp.broadcast_to(ss, ss_ref.shape)

    xc = x.astype(jnp.bfloat16)
    xb_ref[...] = jnp.zeros(xb_ref.shape, jnp.bfloat16)
    for h in range(H):
        xb_ref[:, (h + 1) * Wp + 1:(h + 1) * Wp + 1 + W] = \
            xc[:, h * W:(h + 1) * W]


def _main_kernel(xb_ref, w1_ref, b1_ref, w2_ref, sh_ref, o_ref, *, H, W, Wp, P):
    C = o_ref.shape[0]
    xb = xb_ref[...]

    x3 = _kw_stack(xb, P)
    o1 = jnp.dot(w1_ref[...], x3, preferred_element_type=jnp.float32)
    c1 = _kh_combine(o1, C, P, Wp)

    q = jax.lax.broadcasted_iota(jnp.int32, (1, P), 1)
    hh = q >> 6
    ww = q & (Wp - 1)
    interior = (hh >= 1) & (hh <= H) & (ww >= 1) & (ww <= W)
    m = interior.astype(jnp.float32)
    y = (_leaky(c1 + b1_ref[...]) * m).astype(jnp.bfloat16)

    y3 = _kw_stack(y, P)
    x2 = jnp.concatenate([y3, xb], axis=0)
    o2 = jnp.dot(w2_ref[...], x2, preferred_element_type=jnp.float32)
    out = _leaky(_kh_combine(o2, C, P, Wp) + sh_ref[...])

    orot = _rot(out, Wp + 1, P)
    for h in range(H):
        o_ref[:, h * W:(h + 1) * W] = orot[:, h * Wp:h * Wp + W]


def kernel(x_nchw, w1, b1, w2, b2, wd, bd, gamma, beta):
    del bd
    x_nchw = x_nchw.astype(jnp.float32)
    N, Cin, H, W = x_nchw.shape
    Cout = w1.shape[0]
    f32 = jnp.float32
    bf16 = jnp.bfloat16
    assert Cin == 64 and Cout == 64, "layout assumes 64 channels"

    Hp = H + 2
    Wp = 64
    P = Hp * Wp
    HW = H * W

    x_flat = x_nchw.reshape(N, Cin, HW)
    wdm = wd.reshape(Cout, Cin)

    w1s = jnp.transpose(w1, (2, 0, 3, 1)).reshape(3 * Cout, 3 * Cin)
    w2s = jnp.transpose(w2, (2, 0, 3, 1)).reshape(3 * Cout, 3 * Cout)

    cparams = pltpu.CompilerParams(
        dimension_semantics=("parallel",),
        vmem_limit_bytes=64 * 1024 * 1024)

    stats1 = functools.partial(_stats_kernel, H=H, W=W, Wp=Wp, P=P)
    xb, sums, sqs = pl.pallas_call(
        stats1,
        out_shape=(jax.ShapeDtypeStruct((N, Cin, P), bf16),
                   jax.ShapeDtypeStruct((N, Cout, 128), f32),
                   jax.ShapeDtypeStruct((N, Cout, 128), f32)),
        grid=(N,),
        in_specs=[
            pl.BlockSpec((None, Cin, HW), lambda n: (n, 0, 0)),
            pl.BlockSpec((Cout, Cin), lambda n: (0, 0)),
        ],
        out_specs=(
            pl.BlockSpec((None, Cin, P), lambda n: (n, 0, 0)),
            pl.BlockSpec((None, Cout, 128), lambda n: (n, 0, 0)),
            pl.BlockSpec((None, Cout, 128), lambda n: (n, 0, 0)),
        ),
        compiler_params=cparams,
        cost_estimate=pl.CostEstimate(
            flops=2 * N * Cin * Cout * HW,
            transcendentals=0,
            bytes_accessed=4 * N * Cin * HW + 2 * N * Cin * P),
    )(x_flat, wdm)

    del sums, sqs
    scale = jnp.ones((Cout,), f32)
    shift = jnp.zeros((Cout, 1), f32)

    wds = wdm * scale[:, None]
    w2e = jnp.zeros((3 * Cout, 3 * Cout + Cin), f32)
    w2e = w2e.at[:, :3 * Cout].set(w2s)
    w2e = w2e.at[Cout:2 * Cout, 3 * Cout:].set(wds)

    main = functools.partial(_main_kernel, H=H, W=W, Wp=Wp, P=P)
    out_flat = pl.pallas_call(
        main,
        out_shape=jax.ShapeDtypeStruct((N, Cout, HW), f32),
        grid=(N,),
        in_specs=[
            pl.BlockSpec((None, Cin, P), lambda n: (n, 0, 0)),
            pl.BlockSpec((3 * Cout, 3 * Cin), lambda n: (0, 0)),
            pl.BlockSpec((Cout, 1), lambda n: (0, 0)),
            pl.BlockSpec((3 * Cout, 3 * Cout + Cin), lambda n: (0, 0)),
            pl.BlockSpec((Cout, 1), lambda n: (0, 0)),
        ],
        out_specs=pl.BlockSpec((None, Cout, HW), lambda n: (n, 0, 0)),
        compiler_params=cparams,
        cost_estimate=pl.CostEstimate(
            flops=2 * N * P * (3 * Cin * 3 * Cout + (3 * Cout + Cin) * 3 * Cout) // 1,
            transcendentals=0,
            bytes_accessed=2 * N * Cin * P + 4 * N * Cout * HW),
    )(xb, w1s.astype(bf16), b1.astype(f32).reshape(Cout, 1),
      w2e.astype(bf16), shift)

    return out_flat.reshape(N, Cout, H, W)


# speedup vs baseline: 1.0454x; 1.0454x over previous
import functools

import jax
import jax.numpy as jnp
from jax.experimental import pallas as pl
from jax.experimental.pallas import tpu as pltpu

NEG_SLOPE = 0.01
BN_EPS = 1e-5


def _leaky(v):
    return jnp.where(v >= 0, v, NEG_SLOPE * v)


def _rot(a, s, p):
    s = s % p
    if s == 0:
        return a
    return jnp.concatenate([a[:, s:], a[:, :s]], axis=1)


def _kw_stack(a, p):
    return jnp.concatenate([_rot(a, -1, p), a, _rot(a, 1, p)], axis=0)


def _fused_kernel(x_ref, w1_ref, w2s_ref, wdm_ref, bn_ref, o_ref,
                  xbs_ref, sacc_ref, ssacc_ref, w2e_ref, shift_ref,
                  *, N, H, W, Wp, P):
    ph = pl.program_id(0)
    i = pl.program_id(1)
    C = 64

    @pl.when(ph == 0)
    def _phase0():
        x = x_ref[...]

        @pl.when(i == 0)
        def _init():
            sacc_ref[...] = jnp.zeros_like(sacc_ref)
            ssacc_ref[...] = jnp.zeros_like(ssacc_ref)

        d = jnp.dot(wdm_ref[...], x, preferred_element_type=jnp.float32)
        s = jnp.sum(d, axis=1, keepdims=True)
        ss = jnp.sum(d * d, axis=1, keepdims=True)
        sacc_ref[...] = sacc_ref[...] + jnp.broadcast_to(s, sacc_ref.shape)
        ssacc_ref[...] = ssacc_ref[...] + jnp.broadcast_to(ss, ssacc_ref.shape)

        xc = x.astype(jnp.bfloat16)
        xbi = xbs_ref.at[i]
        xbi[...] = jnp.zeros((C, P), jnp.bfloat16)
        for h in range(H):
            xbi[:, (h + 1) * Wp + 1:(h + 1) * Wp + 1 + W] = \
                xc[:, h * W:(h + 1) * W]

    @pl.when(ph == 1)
    def _phase1():
        @pl.when(i == 0)
        def _finalize_stats():
            inv_cnt = 1.0 / float(N * H * W)
            s = sacc_ref[:, 0:1]
            ss = ssacc_ref[:, 0:1]
            mean = s * inv_cnt
            var = jnp.maximum(ss * inv_cnt - mean * mean, 0.0)
            gamma = bn_ref[:, 1:2]
            beta = bn_ref[:, 2:3]
            b2 = bn_ref[:, 3:4]
            scale = gamma * jax.lax.rsqrt(var + BN_EPS)
            shift_ref[...] = jnp.broadcast_to(
                beta + b2 - mean * scale, shift_ref.shape)
            w2e_ref[...] = jnp.zeros_like(w2e_ref)
            w2e_ref[:, :3 * C] = w2s_ref[...]
            w2e_ref[C:2 * C, 3 * C:] = \
                (wdm_ref[...] * scale).astype(jnp.bfloat16)

        xb = xbs_ref[i]

        x3 = _kw_stack(xb, P)
        o1 = jnp.dot(w1_ref[...], x3, preferred_element_type=jnp.float32)
        c1 = _rot(o1[:C], -Wp, P) + o1[C:2 * C] + _rot(o1[2 * C:], Wp, P)

        q = jax.lax.broadcasted_iota(jnp.int32, (1, P), 1)
        hh = q >> 6
        ww = q & (Wp - 1)
        interior = (hh >= 1) & (hh <= H) & (ww >= 1) & (ww <= W)
        m = interior.astype(jnp.float32)
        b1 = bn_ref[:, 0:1]
        y = (_leaky(c1 + b1) * m).astype(jnp.bfloat16)

        y3 = _kw_stack(y, P)
        x2 = jnp.concatenate([y3, xb], axis=0)
        o2 = jnp.dot(w2e_ref[...], x2, preferred_element_type=jnp.float32)
        o2c = (_rot(o2[:C], 1, P) + _rot(o2[C:2 * C], Wp + 1, P)
               + _rot(o2[2 * C:], 2 * Wp + 1, P))
        out = _leaky(o2c + shift_ref[:, 0:1])
        for h in range(H):
            o_ref[:, h * W:(h + 1) * W] = out[:, h * Wp:h * Wp + W]


def kernel(x_nchw, w1, b1, w2, b2, wd, bd, gamma, beta):
    del bd
    x_nchw = x_nchw.astype(jnp.float32)
    N, Cin, H, W = x_nchw.shape
    Cout = w1.shape[0]
    f32 = jnp.float32
    bf16 = jnp.bfloat16
    assert Cin == 64 and Cout == 64, "layout assumes 64 channels"

    Hp = H + 2
    Wp = 64
    P = Hp * Wp
    HW = H * W

    x_flat = x_nchw.reshape(N, Cin, HW)
    wdm = wd.reshape(Cout, Cin)

    w1s = jnp.transpose(w1, (2, 0, 3, 1)).reshape(3 * Cout, 3 * Cin)
    w2s = jnp.transpose(w2, (2, 0, 3, 1)).reshape(3 * Cout, 3 * Cout)
    bnmat = jnp.stack([b1, gamma, beta, b2], axis=1).astype(f32)

    fused = functools.partial(_fused_kernel, N=N, H=H, W=W, Wp=Wp, P=P)
    out_flat = pl.pallas_call(
        fused,
        out_shape=jax.ShapeDtypeStruct((N, Cout, HW), f32),
        grid=(2, N),
        in_specs=[
            pl.BlockSpec((None, Cin, HW),
                         lambda p, i: (jnp.where(p == 0, i, 0), 0, 0)),
            pl.BlockSpec((3 * Cout, 3 * Cin), lambda p, i: (0, 0)),
            pl.BlockSpec((3 * Cout, 3 * Cout), lambda p, i: (0, 0)),
            pl.BlockSpec((Cout, Cin), lambda p, i: (0, 0)),
            pl.BlockSpec((Cout, 4), lambda p, i: (0, 0)),
        ],
        out_specs=pl.BlockSpec((None, Cout, HW),
                               lambda p, i: (jnp.where(p == 1, i, 0), 0, 0)),
        scratch_shapes=[
            pltpu.VMEM((N, Cin, P), bf16),
            pltpu.VMEM((Cout, 128), f32),
            pltpu.VMEM((Cout, 128), f32),
            pltpu.VMEM((3 * Cout, 3 * Cout + Cin), bf16),
            pltpu.VMEM((Cout, 128), f32),
        ],
        compiler_params=pltpu.CompilerParams(
            dimension_semantics=("arbitrary", "arbitrary"),
            vmem_limit_bytes=64 * 1024 * 1024),
        cost_estimate=pl.CostEstimate(
            flops=2 * N * P * (3 * Cin * 3 * Cout + (3 * Cout + Cin) * 3 * Cout)
            + 2 * N * Cin * Cout * HW,
            transcendentals=0,
            bytes_accessed=4 * N * Cin * HW + 4 * N * Cout * HW),
    )(x_flat, w1s.astype(bf16), w2s.astype(bf16), wdm, bnmat)

    return out_flat.reshape(N, Cout, H, W)
